# Initial kernel scaffold; baseline (speedup 1.0000x reference)
import jax
import jax.numpy as jnp
from jax.experimental import pallas as pl


def kernel(m1_w, m1_b, bn1_gamma, bn1_beta, bn1_mean, bn1_var, m2_w, m2_b, bn2_gamma, bn2_beta, bn2_mean, bn2_var, m3_w, m3_b, bn3_gamma, bn3_beta, bn3_mean, bn3_var, m4_w, m4_b, bn4_gamma, bn4_beta, bn4_mean, bn4_var, r_w1, r_b1, r_w2, r_b2, weight_pool, x):
    raise NotImplementedError("write your pallas kernel here")


# speedup vs baseline: 1.5963x; 1.5963x over previous
import functools

import jax
import jax.numpy as jnp
import numpy as np
from jax.experimental import pallas as pl
from jax.experimental.pallas import tpu as pltpu

_EPS = 1e-5
_BF = jnp.bfloat16


def _rup(v, m):
    return ((v + m - 1) // m) * m


def _fold_conv_bn(w, b, gamma, beta, mean, var):
    s = gamma * jax.lax.rsqrt(var + _EPS)
    wf = w * s[:, None, None, None]
    bf = (b - mean) * s + beta
    co, ci = wf.shape[0], wf.shape[1]
    return jnp.transpose(wf, (0, 2, 3, 1)).reshape(co, 9 * ci), bf


def _up_mat(n_in):
    u = np.zeros((2 * n_in, n_in), np.float32)
    for o in range(2 * n_in):
        src = max((o + 0.5) * 0.5 - 0.5, 0.0)
        i0 = int(np.floor(src))
        f = src - i0
        u[o, i0] += 1.0 - f
        u[o, min(i0 + 1, n_in - 1)] += f
    return u


def _mask_mats(h, w):
    h2, w2, wp = h // 2, w // 2, w + 2
    uhp = np.zeros((h + 2, h2), np.float32)
    uhp[1:h + 1] = _up_mat(h2)
    uwp = np.zeros((w + 2, w2), np.float32)
    uwp[1:w + 1] = _up_mat(w2)
    sh = np.zeros((9, h, h2), np.float32)
    sw = np.zeros((9, w2, wp), np.float32)
    for kh in range(3):
        for kw in range(3):
            t = 3 * kh + kw
            sh[t] = uhp[kh:kh + h]
            sw[t][:, 1:1 + w] = uwp[kw:kw + w].T
    return jnp.asarray(sh), jnp.asarray(sw)


def _taps(src, wp, n):
    return jnp.concatenate(
        [src((kh + 1) * wp + kw - 1, n) for kh in range(3) for kw in range(3)],
        axis=0)


def _pad_lanes(v, total):
    cur = v.shape[-1]
    if total == cur:
        return v
    return jnp.concatenate(
        [v, jnp.zeros(v.shape[:-1] + (total - cur,), v.dtype)], axis=-1)


def _branch_body(xs_ref, w1_ref, b1_ref, w2_ref, b2_ref, w3_ref, b3_ref,
                 vm_ref, o_ref, *, c, wp, lh, ls_pad):
    wph = wp // 2
    xsb = xs_ref[...]
    vm = vm_ref[...]
    acc = jnp.zeros((16, lh), jnp.float32)
    for a in (0, 1):
        for b in (0, 1):
            parts = []
            for kh in range(3):
                for kw in range(3):
                    par = (b + kw) % 2
                    off = (a + kh + 1) * wph + (b + kw) // 2
                    parts.append(xsb[par * c:par * c + c, off:off + lh])
            st1 = jnp.concatenate(parts, axis=0)
            y1 = jnp.dot(w1_ref[...], st1,
                         preferred_element_type=jnp.float32) + b1_ref[...]
            acc = acc + jnp.maximum(y1, 0.0)
    pooled = (0.25 * acc) * vm
    zl = jnp.zeros((16, 2 * wp), jnp.float32)
    zt = jnp.zeros((16, ls_pad - 2 * wp - lh), jnp.float32)
    ext1 = jnp.concatenate([zl, pooled, zt], axis=1).astype(_BF)
    st2 = _taps(lambda o, m: ext1[:, o:o + m], wp, lh)
    y2 = jnp.dot(w2_ref[...], st2, preferred_element_type=jnp.float32)
    y2 = jnp.maximum(y2 + b2_ref[...], 0.0) * vm
    ext2 = jnp.concatenate([zl, y2, zt], axis=1).astype(_BF)
    st3 = _taps(lambda o, m: ext2[:, o:o + m], wp, lh)
    y3 = jnp.dot(w3_ref[...], st3, preferred_element_type=jnp.float32)
    y3 = jnp.maximum(y3 + b3_ref[...], 0.0)
    o_ref[...] = _pad_lanes(y3, o_ref.shape[-1])


def _adapt_body(xe_ref, wa_ref, mhat_ref, sh_ref, sw_ref, b4_ref, o_ref, *,
                c, wp, h, core):
    h2 = h // 2
    acc = jnp.zeros((h, wp), jnp.float32)
    for t in range(9):
        tmp = jnp.dot(mhat_ref[t * h2:(t + 1) * h2, :], sw_ref[t],
                      preferred_element_type=jnp.float32)
        acc = acc + jnp.dot(sh_ref[t], tmp,
                            preferred_element_type=jnp.float32)
    mask2d = jax.nn.sigmoid(acc + b4_ref[...])
    mask_flat = jnp.concatenate([mask2d[r:r + 1, :] for r in range(h)],
                                axis=1)
    xeb = xe_ref[...]
    st = _taps(lambda o, m: xeb[:, o:o + m], wp, core)
    adapted = jnp.dot(wa_ref[...], st, preferred_element_type=jnp.float32)
    res = xeb[:, 2 * wp:2 * wp + core].astype(jnp.float32) \
        + adapted * mask_flat
    o_ref[...] = _pad_lanes(res, o_ref.shape[-1])


def kernel(m1_w, m1_b, bn1_gamma, bn1_beta, bn1_mean, bn1_var,
           m2_w, m2_b, bn2_gamma, bn2_beta, bn2_mean, bn2_var,
           m3_w, m3_b, bn3_gamma, bn3_beta, bn3_mean, bn3_var,
           m4_w, m4_b, bn4_gamma, bn4_beta, bn4_mean, bn4_var,
           r_w1, r_b1, r_w2, r_b2, weight_pool, x):
    x = x.astype(jnp.float32)
    n, c, h, w = x.shape
    h2, w2 = h // 2, w // 2
    wp = w + 2
    wph = wp // 2
    core = h * wp
    lh = h2 * wp
    core_pad = _rup(core, 128)
    le_pad = _rup((h + 4) * wp, 128)
    ls_pad = _rup((h2 + 4) * wp, 128)
    lsp_pad = _rup((h + 4) * wph + 1, 128)
    lmid_pad = _rup(lh, 128)

    w1s, b1 = _fold_conv_bn(m1_w, m1_b, bn1_gamma, bn1_beta, bn1_mean, bn1_var)
    w2s, b2 = _fold_conv_bn(m2_w, m2_b, bn2_gamma, bn2_beta, bn2_mean, bn2_var)
    w3s, b3 = _fold_conv_bn(m3_w, m3_b, bn3_gamma, bn3_beta, bn3_mean, bn3_var)
    s4 = bn4_gamma * jax.lax.rsqrt(bn4_var + _EPS)
    w4f = m4_w * s4[:, None, None, None]
    b4 = (m4_b - bn4_mean) * s4 + bn4_beta
    w4m = jnp.transpose(w4f[0], (1, 2, 0)).reshape(9, 16)

    s = jnp.array([[1.0 / 2.0, 1.0 / 1.5]], jnp.float32)
    hid = jax.nn.relu(s @ r_w1 + r_b1)
    rw = jax.nn.softmax(hid @ r_w2 + r_b2, axis=-1)[0]
    fused = jnp.einsum("e,eoikl->oikl", rw, weight_pool)
    was = jnp.transpose(fused, (0, 2, 3, 1)).reshape(c, 9 * c).astype(_BF)

    xb = x.astype(_BF)
    xpad = jnp.pad(xb, ((0, 0), (0, 0), (2, 2), (1, 1)))
    xe = xpad.reshape(n, c, (h + 4) * wp)
    xe = jnp.pad(xe, ((0, 0), (0, 0), (0, le_pad - (h + 4) * wp)))
    xsp = xpad.reshape(n, c, h + 4, wph, 2)
    xs = jnp.concatenate([xsp[..., 0], xsp[..., 1]], axis=1)
    xs = xs.reshape(n, 2 * c, (h + 4) * wph)
    xs = jnp.pad(xs, ((0, 0), (0, 0), (0, lsp_pad - (h + 4) * wph)))

    vm_np = np.zeros((h2, wp), np.float32)
    vm_np[:, :w2] = 1.0
    vm = jnp.asarray(vm_np.reshape(1, lh))
    sh, sw = _mask_mats(h, w)

    cparams = pltpu.CompilerParams(dimension_semantics=("parallel",),
                                   vmem_limit_bytes=64 * 1024 * 1024)

    mid = pl.pallas_call(
        functools.partial(_branch_body, c=c, wp=wp, lh=lh, ls_pad=ls_pad),
        out_shape=jax.ShapeDtypeStruct((n, 16, lmid_pad), jnp.float32),
        grid=(n,),
        in_specs=[
            pl.BlockSpec((None, 2 * c, lsp_pad), lambda i: (i, 0, 0)),
            pl.BlockSpec((16, 9 * c), lambda i: (0, 0)),
            pl.BlockSpec((16, 1), lambda i: (0, 0)),
            pl.BlockSpec((16, 144), lambda i: (0, 0)),
            pl.BlockSpec((16, 1), lambda i: (0, 0)),
            pl.BlockSpec((16, 144), lambda i: (0, 0)),
            pl.BlockSpec((16, 1), lambda i: (0, 0)),
            pl.BlockSpec((1, lh), lambda i: (0, 0)),
        ],
        out_specs=pl.BlockSpec((None, 16, lmid_pad), lambda i: (i, 0, 0)),
        compiler_params=cparams,
    )(xs, w1s.astype(_BF), b1.reshape(16, 1), w2s.astype(_BF),
      b2.reshape(16, 1), w3s.astype(_BF), b3.reshape(16, 1), vm)

    feat = mid[:, :, :lh].reshape(n, 16, h2, wp)[:, :, :, :w2]
    mhat = jnp.einsum("tc,nchw->nthw", w4m, feat).reshape(n, 9 * h2, w2)

    outslab = pl.pallas_call(
        functools.partial(_adapt_body, c=c, wp=wp, h=h, core=core),
        out_shape=jax.ShapeDtypeStruct((n, c, core_pad), jnp.float32),
        grid=(n,),
        in_specs=[
            pl.BlockSpec((None, c, le_pad), lambda i: (i, 0, 0)),
            pl.BlockSpec((c, 9 * c), lambda i: (0, 0)),
            pl.BlockSpec((None, 9 * h2, w2), lambda i: (i, 0, 0)),
            pl.BlockSpec((9, h, h2), lambda i: (0, 0, 0)),
            pl.BlockSpec((9, w2, wp), lambda i: (0, 0, 0)),
            pl.BlockSpec((1, 1), lambda i: (0, 0)),
        ],
        out_specs=pl.BlockSpec((None, c, core_pad), lambda i: (i, 0, 0)),
        compiler_params=cparams,
    )(xe, was, mhat, sh, sw, b4.reshape(1, 1))

    return outslab[:, :, :core].reshape(n, c, h, wp)[:, :, :, 1:1 + w]


# speedup vs baseline: 1.8408x; 1.1531x over previous
import functools

import jax
import jax.numpy as jnp
import numpy as np
from jax.experimental import pallas as pl
from jax.experimental.pallas import tpu as pltpu

_EPS = 1e-5
_BF = jnp.bfloat16


def _rup(v, m):
    return ((v + m - 1) // m) * m


def _fold_conv_bn(w, b, gamma, beta, mean, var):
    s = gamma * jax.lax.rsqrt(var + _EPS)
    wf = w * s[:, None, None, None]
    bf = (b - mean) * s + beta
    co, ci = wf.shape[0], wf.shape[1]
    return jnp.transpose(wf, (0, 2, 3, 1)).reshape(co, 9 * ci), bf


def _up_mat(n_in):
    u = np.zeros((2 * n_in, n_in), np.float32)
    for o in range(2 * n_in):
        src = max((o + 0.5) * 0.5 - 0.5, 0.0)
        i0 = int(np.floor(src))
        f = src - i0
        u[o, i0] += 1.0 - f
        u[o, min(i0 + 1, n_in - 1)] += f
    return u


def _mask_mats(h, w):
    h2, w2, wp = h // 2, w // 2, w + 2
    uhp = np.zeros((h + 2, h2), np.float32)
    uhp[1:h + 1] = _up_mat(h2)
    uwp = np.zeros((w + 2, w2), np.float32)
    uwp[1:w + 1] = _up_mat(w2)
    sh = np.zeros((9, h, h2), np.float32)
    sw = np.zeros((9, w2, wp), np.float32)
    for kh in range(3):
        for kw in range(3):
            t = 3 * kh + kw
            sh[t] = uhp[kh:kh + h]
            sw[t][:, 1:1 + w] = uwp[kw:kw + w].T
    return jnp.asarray(sh), jnp.asarray(sw)


def _taps(src, wp, n):
    return jnp.concatenate(
        [src((kh + 1) * wp + kw - 1, n) for kh in range(3) for kw in range(3)],
        axis=0)


def _pad_lanes(v, total):
    cur = v.shape[-1]
    if total == cur:
        return v
    return jnp.concatenate(
        [v, jnp.zeros(v.shape[:-1] + (total - cur,), v.dtype)], axis=-1)


def _branch_body(xs_ref, w1_ref, b1_ref, w2_ref, b2_ref, w3_ref, b3_ref,
                 vm_ref, o_ref, *, c, wp, lh, ls_pad):
    wph = wp // 2
    xsb = xs_ref[...]
    vm = vm_ref[...]
    acc = jnp.zeros((16, lh), jnp.float32)
    for a in (0, 1):
        for b in (0, 1):
            parts = []
            for kh in range(3):
                for kw in range(3):
                    par = (b + kw) % 2
                    off = (a + kh + 1) * wph + (b + kw) // 2
                    parts.append(xsb[par * c:par * c + c, off:off + lh])
            st1 = jnp.concatenate(parts, axis=0)
            y1 = jnp.dot(w1_ref[...], st1,
                         preferred_element_type=jnp.float32) + b1_ref[...]
            acc = acc + jnp.maximum(y1, 0.0)
    pooled = (0.25 * acc) * vm
    zl = jnp.zeros((16, 2 * wp), jnp.float32)
    zt = jnp.zeros((16, ls_pad - 2 * wp - lh), jnp.float32)
    ext1 = jnp.concatenate([zl, pooled, zt], axis=1).astype(_BF)
    st2 = _taps(lambda o, m: ext1[:, o:o + m], wp, lh)
    y2 = jnp.dot(w2_ref[...], st2, preferred_element_type=jnp.float32)
    y2 = jnp.maximum(y2 + b2_ref[...], 0.0) * vm
    ext2 = jnp.concatenate([zl, y2, zt], axis=1).astype(_BF)
    st3 = _taps(lambda o, m: ext2[:, o:o + m], wp, lh)
    y3 = jnp.dot(w3_ref[...], st3, preferred_element_type=jnp.float32)
    y3 = jnp.maximum(y3 + b3_ref[...], 0.0)
    o_ref[...] = _pad_lanes(y3, o_ref.shape[-1])


def _adapt_body(x_ref, wa_ref, mhat_ref, sh_ref, sw_ref, b4_ref, o_ref,
                ext_ref, *, c, wp, h, w, core):
    h2 = h // 2
    acc = jnp.zeros((h, wp), jnp.float32)
    for t in range(9):
        tmp = jnp.dot(mhat_ref[t * h2:(t + 1) * h2, :], sw_ref[t],
                      preferred_element_type=jnp.float32)
        acc = acc + jnp.dot(sh_ref[t], tmp,
                            preferred_element_type=jnp.float32)
    mask2d = jax.nn.sigmoid(acc + b4_ref[...])
    mask_flat = jnp.concatenate([mask2d[r:r + 1, :] for r in range(h)],
                                axis=1)
    ext_ref[...] = jnp.zeros(ext_ref.shape, _BF)
    xb = x_ref[...].astype(_BF)
    for r in range(h):
        base = 2 * wp + r * wp + 1
        ext_ref[:, base:base + w] = xb[:, r * w:(r + 1) * w]
    st = _taps(lambda o, m: ext_ref[:, o:o + m], wp, core)
    adapted = jnp.dot(wa_ref[...], st, preferred_element_type=jnp.float32)
    am = adapted * mask_flat
    for r in range(h):
        o_ref[:, r * w:(r + 1) * w] = x_ref[:, r * w:(r + 1) * w] \
            + am[:, r * wp + 1:r * wp + 1 + w]


def kernel(m1_w, m1_b, bn1_gamma, bn1_beta, bn1_mean, bn1_var,
           m2_w, m2_b, bn2_gamma, bn2_beta, bn2_mean, bn2_var,
           m3_w, m3_b, bn3_gamma, bn3_beta, bn3_mean, bn3_var,
           m4_w, m4_b, bn4_gamma, bn4_beta, bn4_mean, bn4_var,
           r_w1, r_b1, r_w2, r_b2, weight_pool, x):
    x = x.astype(jnp.float32)
    n, c, h, w = x.shape
    h2, w2 = h // 2, w // 2
    wp = w + 2
    wph = wp // 2
    core = h * wp
    lh = h2 * wp
    core_pad = _rup(core, 128)
    le_pad = _rup((h + 4) * wp, 128)
    ls_pad = _rup((h2 + 4) * wp, 128)
    lsp_pad = _rup((h + 4) * wph + 1, 128)
    lmid_pad = _rup(lh, 128)

    w1s, b1 = _fold_conv_bn(m1_w, m1_b, bn1_gamma, bn1_beta, bn1_mean, bn1_var)
    w2s, b2 = _fold_conv_bn(m2_w, m2_b, bn2_gamma, bn2_beta, bn2_mean, bn2_var)
    w3s, b3 = _fold_conv_bn(m3_w, m3_b, bn3_gamma, bn3_beta, bn3_mean, bn3_var)
    s4 = bn4_gamma * jax.lax.rsqrt(bn4_var + _EPS)
    w4f = m4_w * s4[:, None, None, None]
    b4 = (m4_b - bn4_mean) * s4 + bn4_beta
    w4m = jnp.transpose(w4f[0], (1, 2, 0)).reshape(9, 16)

    s = jnp.array([[1.0 / 2.0, 1.0 / 1.5]], jnp.float32)
    hid = jax.nn.relu(s @ r_w1 + r_b1)
    rw = jax.nn.softmax(hid @ r_w2 + r_b2, axis=-1)[0]
    fused = jnp.einsum("e,eoikl->oikl", rw, weight_pool)
    was = jnp.transpose(fused, (0, 2, 3, 1)).reshape(c, 9 * c).astype(_BF)

    xb = x.astype(_BF)
    xpad = jnp.pad(xb, ((0, 0), (0, 0), (2, 2), (1, 1)))
    xsp = xpad.reshape(n, c, h + 4, wph, 2)
    xs = jnp.concatenate([xsp[..., 0], xsp[..., 1]], axis=1)
    xs = xs.reshape(n, 2 * c, (h + 4) * wph)
    xs = jnp.pad(xs, ((0, 0), (0, 0), (0, lsp_pad - (h + 4) * wph)))

    vm_np = np.zeros((h2, wp), np.float32)
    vm_np[:, :w2] = 1.0
    vm = jnp.asarray(vm_np.reshape(1, lh))
    sh, sw = _mask_mats(h, w)

    cparams = pltpu.CompilerParams(dimension_semantics=("parallel",),
                                   vmem_limit_bytes=64 * 1024 * 1024)

    mid = pl.pallas_call(
        functools.partial(_branch_body, c=c, wp=wp, lh=lh, ls_pad=ls_pad),
        out_shape=jax.ShapeDtypeStruct((n, 16, lmid_pad), jnp.float32),
        grid=(n,),
        in_specs=[
            pl.BlockSpec((None, 2 * c, lsp_pad), lambda i: (i, 0, 0)),
            pl.BlockSpec((16, 9 * c), lambda i: (0, 0)),
            pl.BlockSpec((16, 1), lambda i: (0, 0)),
            pl.BlockSpec((16, 144), lambda i: (0, 0)),
            pl.BlockSpec((16, 1), lambda i: (0, 0)),
            pl.BlockSpec((16, 144), lambda i: (0, 0)),
            pl.BlockSpec((16, 1), lambda i: (0, 0)),
            pl.BlockSpec((1, lh), lambda i: (0, 0)),
        ],
        out_specs=pl.BlockSpec((None, 16, lmid_pad), lambda i: (i, 0, 0)),
        compiler_params=cparams,
    )(xs, w1s.astype(_BF), b1.reshape(16, 1), w2s.astype(_BF),
      b2.reshape(16, 1), w3s.astype(_BF), b3.reshape(16, 1), vm)

    feat = mid[:, :, :lh].reshape(n, 16, h2, wp)[:, :, :, :w2]
    mhat = jnp.einsum("tc,nchw->nthw", w4m, feat).reshape(n, 9 * h2, w2)

    outflat = pl.pallas_call(
        functools.partial(_adapt_body, c=c, wp=wp, h=h, w=w, core=core),
        out_shape=jax.ShapeDtypeStruct((n, c, h * w), jnp.float32),
        grid=(n,),
        in_specs=[
            pl.BlockSpec((None, c, h * w), lambda i: (i, 0, 0)),
            pl.BlockSpec((c, 9 * c), lambda i: (0, 0)),
            pl.BlockSpec((None, 9 * h2, w2), lambda i: (i, 0, 0)),
            pl.BlockSpec((9, h, h2), lambda i: (0, 0, 0)),
            pl.BlockSpec((9, w2, wp), lambda i: (0, 0, 0)),
            pl.BlockSpec((1, 1), lambda i: (0, 0)),
        ],
        out_specs=pl.BlockSpec((None, c, h * w), lambda i: (i, 0, 0)),
        scratch_shapes=[pltpu.VMEM((c, _rup(core + 4 * wp, 128)), _BF)],
        compiler_params=cparams,
    )(x.reshape(n, c, h * w), was, mhat, sh, sw, b4.reshape(1, 1))

    return outflat.reshape(n, c, h, w)


# speedup vs baseline: 1.8840x; 1.0235x over previous
import functools

import jax
import jax.numpy as jnp
import numpy as np
from jax.experimental import pallas as pl
from jax.experimental.pallas import tpu as pltpu

_EPS = 1e-5
_BF = jnp.bfloat16


def _rup(v, m):
    return ((v + m - 1) // m) * m


def _fold_conv_bn(w, b, gamma, beta, mean, var):
    s = gamma * jax.lax.rsqrt(var + _EPS)
    wf = w * s[:, None, None, None]
    bf = (b - mean) * s + beta
    co, ci = wf.shape[0], wf.shape[1]
    return jnp.transpose(wf, (0, 2, 3, 1)).reshape(co, 9 * ci), bf


def _up_mat(n_in):
    u = np.zeros((2 * n_in, n_in), np.float32)
    for o in range(2 * n_in):
        src = max((o + 0.5) * 0.5 - 0.5, 0.0)
        i0 = int(np.floor(src))
        f = src - i0
        u[o, i0] += 1.0 - f
        u[o, min(i0 + 1, n_in - 1)] += f
    return u


def _mask_mats(h, w):
    h2, w2, wp = h // 2, w // 2, w + 2
    uhp = np.zeros((h + 2, h2), np.float32)
    uhp[1:h + 1] = _up_mat(h2)
    uwp = np.zeros((w + 2, w2), np.float32)
    uwp[1:w + 1] = _up_mat(w2)
    sh = np.zeros((9, h, h2), np.float32)
    sw = np.zeros((9, w2, wp), np.float32)
    for kh in range(3):
        for kw in range(3):
            t = 3 * kh + kw
            sh[t] = uhp[kh:kh + h]
            sw[t][:, 1:1 + w] = uwp[kw:kw + w].T
    return jnp.asarray(sh), jnp.asarray(sw)


def _taps(src, wp, n):
    return jnp.concatenate(
        [src((kh + 1) * wp + kw - 1, n) for kh in range(3) for kw in range(3)],
        axis=0)


def _pad_lanes(v, total):
    cur = v.shape[-1]
    if total == cur:
        return v
    return jnp.concatenate(
        [v, jnp.zeros(v.shape[:-1] + (total - cur,), v.dtype)], axis=-1)


def _branch_body(xs_ref, w1_ref, b1_ref, w2_ref, b2_ref, w3_ref, b3_ref,
                 w4_ref, vm_ref, o_ref, *, c, wp, lh, ls_pad):
    wph = wp // 2
    xsb = xs_ref[...]
    vm = vm_ref[...]
    acc = jnp.zeros((16, lh), jnp.float32)
    for a in (0, 1):
        for b in (0, 1):
            parts = []
            for kh in range(3):
                for kw in range(3):
                    par = (b + kw) % 2
                    off = (a + kh + 1) * wph + (b + kw) // 2
                    parts.append(xsb[par * c:par * c + c, off:off + lh])
            st1 = jnp.concatenate(parts, axis=0)
            y1 = jnp.dot(w1_ref[...], st1,
                         preferred_element_type=jnp.float32) + b1_ref[...]
            acc = acc + jnp.maximum(y1, 0.0)
    pooled = (0.25 * acc) * vm
    zl = jnp.zeros((16, 2 * wp), jnp.float32)
    zt = jnp.zeros((16, ls_pad - 2 * wp - lh), jnp.float32)
    ext1 = jnp.concatenate([zl, pooled, zt], axis=1).astype(_BF)
    st2 = _taps(lambda o, m: ext1[:, o:o + m], wp, lh)
    y2 = jnp.dot(w2_ref[...], st2, preferred_element_type=jnp.float32)
    y2 = jnp.maximum(y2 + b2_ref[...], 0.0) * vm
    ext2 = jnp.concatenate([zl, y2, zt], axis=1).astype(_BF)
    st3 = _taps(lambda o, m: ext2[:, o:o + m], wp, lh)
    y3 = jnp.dot(w3_ref[...], st3, preferred_element_type=jnp.float32)
    y3 = jnp.maximum(y3 + b3_ref[...], 0.0)
    mh = jnp.dot(w4_ref[...], y3.astype(_BF),
                 preferred_element_type=jnp.float32)
    o_ref[...] = _pad_lanes(mh.astype(_BF), o_ref.shape[-1])


def _adapt_body(x_ref, wa_ref, mhat_ref, sh_ref, sw_ref, b4_ref, o_ref,
                ext_ref, *, c, wp, h, w, core):
    h2 = h // 2
    acc = jnp.zeros((h, wp), jnp.float32)
    for t in range(9):
        tmp = jnp.dot(mhat_ref[t * h2:(t + 1) * h2, :], sw_ref[t],
                      preferred_element_type=jnp.float32)
        acc = acc + jnp.dot(sh_ref[t], tmp,
                            preferred_element_type=jnp.float32)
    mask2d = jax.nn.sigmoid(acc + b4_ref[...])
    mask_flat = jnp.concatenate([mask2d[r:r + 1, :] for r in range(h)],
                                axis=1)
    ext_ref[...] = jnp.zeros(ext_ref.shape, _BF)
    xb = x_ref[...].astype(_BF)
    for r in range(h):
        base = 2 * wp + r * wp + 1
        ext_ref[:, base:base + w] = xb[:, r * w:(r + 1) * w]
    st = _taps(lambda o, m: ext_ref[:, o:o + m], wp, core)
    adapted = jnp.dot(wa_ref[...], st, preferred_element_type=jnp.float32)
    am = adapted * mask_flat
    for r in range(h):
        o_ref[:, r * w:(r + 1) * w] = x_ref[:, r * w:(r + 1) * w] \
            + am[:, r * wp + 1:r * wp + 1 + w]


def kernel(m1_w, m1_b, bn1_gamma, bn1_beta, bn1_mean, bn1_var,
           m2_w, m2_b, bn2_gamma, bn2_beta, bn2_mean, bn2_var,
           m3_w, m3_b, bn3_gamma, bn3_beta, bn3_mean, bn3_var,
           m4_w, m4_b, bn4_gamma, bn4_beta, bn4_mean, bn4_var,
           r_w1, r_b1, r_w2, r_b2, weight_pool, x):
    x = x.astype(jnp.float32)
    n, c, h, w = x.shape
    h2, w2 = h // 2, w // 2
    wp = w + 2
    wph = wp // 2
    core = h * wp
    lh = h2 * wp
    core_pad = _rup(core, 128)
    le_pad = _rup((h + 4) * wp, 128)
    ls_pad = _rup((h2 + 4) * wp, 128)
    lsp_pad = _rup((h + 4) * wph + 1, 128)
    lmid_pad = _rup(lh, 128)

    w1s, b1 = _fold_conv_bn(m1_w, m1_b, bn1_gamma, bn1_beta, bn1_mean, bn1_var)
    w2s, b2 = _fold_conv_bn(m2_w, m2_b, bn2_gamma, bn2_beta, bn2_mean, bn2_var)
    w3s, b3 = _fold_conv_bn(m3_w, m3_b, bn3_gamma, bn3_beta, bn3_mean, bn3_var)
    s4 = bn4_gamma * jax.lax.rsqrt(bn4_var + _EPS)
    w4f = m4_w * s4[:, None, None, None]
    b4 = (m4_b - bn4_mean) * s4 + bn4_beta
    w4m = jnp.transpose(w4f[0], (1, 2, 0)).reshape(9, 16)
    w4p = jnp.zeros((16, 16), jnp.float32).at[:9].set(w4m).astype(_BF)

    s = jnp.array([[1.0 / 2.0, 1.0 / 1.5]], jnp.float32)
    hid = jax.nn.relu(s @ r_w1 + r_b1)
    rw = jax.nn.softmax(hid @ r_w2 + r_b2, axis=-1)[0]
    fused = jnp.einsum("e,eoikl->oikl", rw, weight_pool)
    was = jnp.transpose(fused, (0, 2, 3, 1)).reshape(c, 9 * c).astype(_BF)

    xb = x.astype(_BF)
    xpad = jnp.pad(xb, ((0, 0), (0, 0), (2, 2), (1, 1)))
    xsp = xpad.reshape(n, c, h + 4, wph, 2)
    xs = jnp.concatenate([xsp[..., 0], xsp[..., 1]], axis=1)
    xs = xs.reshape(n, 2 * c, (h + 4) * wph)
    xs = jnp.pad(xs, ((0, 0), (0, 0), (0, lsp_pad - (h + 4) * wph)))

    vm_np = np.zeros((h2, wp), np.float32)
    vm_np[:, :w2] = 1.0
    vm = jnp.asarray(vm_np.reshape(1, lh))
    sh, sw = _mask_mats(h, w)

    cparams = pltpu.CompilerParams(dimension_semantics=("parallel",),
                                   vmem_limit_bytes=64 * 1024 * 1024)

    mh = pl.pallas_call(
        functools.partial(_branch_body, c=c, wp=wp, lh=lh, ls_pad=ls_pad),
        out_shape=jax.ShapeDtypeStruct((n, 16, lmid_pad), _BF),
        grid=(n,),
        in_specs=[
            pl.BlockSpec((None, 2 * c, lsp_pad), lambda i: (i, 0, 0)),
            pl.BlockSpec((16, 9 * c), lambda i: (0, 0)),
            pl.BlockSpec((16, 1), lambda i: (0, 0)),
            pl.BlockSpec((16, 144), lambda i: (0, 0)),
            pl.BlockSpec((16, 1), lambda i: (0, 0)),
            pl.BlockSpec((16, 144), lambda i: (0, 0)),
            pl.BlockSpec((16, 1), lambda i: (0, 0)),
            pl.BlockSpec((16, 16), lambda i: (0, 0)),
            pl.BlockSpec((1, lh), lambda i: (0, 0)),
        ],
        out_specs=pl.BlockSpec((None, 16, lmid_pad), lambda i: (i, 0, 0)),
        compiler_params=cparams,
    )(xs, w1s.astype(_BF), b1.reshape(16, 1), w2s.astype(_BF),
      b2.reshape(16, 1), w3s.astype(_BF), b3.reshape(16, 1), w4p, vm)

    mhat = (mh[:, :9, :lh].reshape(n, 9, h2, wp)[:, :, :, :w2]
            .reshape(n, 9 * h2, w2).astype(jnp.float32))

    outflat = pl.pallas_call(
        functools.partial(_adapt_body, c=c, wp=wp, h=h, w=w, core=core),
        out_shape=jax.ShapeDtypeStruct((n, c, h * w), jnp.float32),
        grid=(n,),
        in_specs=[
            pl.BlockSpec((None, c, h * w), lambda i: (i, 0, 0)),
            pl.BlockSpec((c, 9 * c), lambda i: (0, 0)),
            pl.BlockSpec((None, 9 * h2, w2), lambda i: (i, 0, 0)),
            pl.BlockSpec((9, h, h2), lambda i: (0, 0, 0)),
            pl.BlockSpec((9, w2, wp), lambda i: (0, 0, 0)),
            pl.BlockSpec((1, 1), lambda i: (0, 0)),
        ],
        out_specs=pl.BlockSpec((None, c, h * w), lambda i: (i, 0, 0)),
        scratch_shapes=[pltpu.VMEM((c, _rup(core + 4 * wp, 128)), _BF)],
        compiler_params=cparams,
    )(x.reshape(n, c, h * w), was, mhat, sh, sw, b4.reshape(1, 1))

    return outflat.reshape(n, c, h, w)


# speedup vs baseline: 2.2578x; 1.1984x over previous
import functools

import jax
import jax.numpy as jnp
import numpy as np
from jax.experimental import pallas as pl
from jax.experimental.pallas import tpu as pltpu

_EPS = 1e-5
_BF = jnp.bfloat16


def _rup(v, m):
    return ((v + m - 1) // m) * m


def _fold_conv_bn(w, b, gamma, beta, mean, var):
    s = gamma * jax.lax.rsqrt(var + _EPS)
    wf = w * s[:, None, None, None]
    bf = (b - mean) * s + beta
    co, ci = wf.shape[0], wf.shape[1]
    return jnp.transpose(wf, (0, 2, 3, 1)).reshape(co, 9 * ci), bf


def _up_mat(n_in):
    u = np.zeros((2 * n_in, n_in), np.float32)
    for o in range(2 * n_in):
        src = max((o + 0.5) * 0.5 - 0.5, 0.0)
        i0 = int(np.floor(src))
        f = src - i0
        u[o, i0] += 1.0 - f
        u[o, min(i0 + 1, n_in - 1)] += f
    return u


def _mask_mats(h, w):
    h2, w2, wp = h // 2, w // 2, w + 2
    uhp = np.zeros((h + 2, h2), np.float32)
    uhp[1:h + 1] = _up_mat(h2)
    uwp = np.zeros((w + 2, w2), np.float32)
    uwp[1:w + 1] = _up_mat(w2)
    sh = np.zeros((9, h, h2), np.float32)
    sw = np.zeros((9, w2, wp), np.float32)
    for kh in range(3):
        for kw in range(3):
            t = 3 * kh + kw
            sh[t] = uhp[kh:kh + h]
            sw[t][:, 1:1 + w] = uwp[kw:kw + w].T
    return jnp.asarray(sh), jnp.asarray(sw)


def _taps(src, wp, n):
    return jnp.concatenate(
        [src((kh + 1) * wp + kw - 1, n) for kh in range(3) for kw in range(3)],
        axis=0)


def _pad_lanes(v, total):
    cur = v.shape[-1]
    if total == cur:
        return v
    return jnp.concatenate(
        [v, jnp.zeros(v.shape[:-1] + (total - cur,), v.dtype)], axis=-1)


def _branch_body(x_ref, w1_ref, b1_ref, w2_ref, b2_ref, w3_ref, b3_ref,
                 w4_ref, vme_ref, o_ref, ext_ref, *, c, wp, h, w, core):
    ext_ref[...] = jnp.zeros(ext_ref.shape, _BF)
    xb = x_ref[...].astype(_BF)
    for r in range(h):
        base = 2 * wp + r * wp + 1
        ext_ref[:, base:base + w] = xb[:, r * w:(r + 1) * w]
    lev = core + wp + 2
    st1 = _taps(lambda o, m: ext_ref[:, o:o + m], wp, lev)
    y1 = jnp.dot(w1_ref[...], st1,
                 preferred_element_type=jnp.float32) + b1_ref[...]
    y1 = jnp.maximum(y1, 0.0)
    vme = vme_ref[...]
    pooled = 0.25 * (y1[:, :core] + y1[:, 1:core + 1]
                     + y1[:, wp:core + wp] + y1[:, wp + 1:core + wp + 1])
    pooled = pooled * vme
    ext2_len = _rup(core + 4 * wp + 8, 128)
    zl = jnp.zeros((16, 2 * wp + 2), jnp.float32)
    zt = jnp.zeros((16, ext2_len - 2 * wp - 2 - core), jnp.float32)
    ext2 = jnp.concatenate([zl, pooled, zt], axis=1).astype(_BF)
    st2 = jnp.concatenate(
        [ext2[:, kh * 2 * wp + 2 * kw:kh * 2 * wp + 2 * kw + core]
         for kh in range(3) for kw in range(3)], axis=0)
    y2 = jnp.dot(w2_ref[...], st2, preferred_element_type=jnp.float32)
    y2 = jnp.maximum(y2 + b2_ref[...], 0.0) * vme
    ext3 = jnp.concatenate([zl, y2, zt], axis=1).astype(_BF)
    st3 = jnp.concatenate(
        [ext3[:, kh * 2 * wp + 2 * kw:kh * 2 * wp + 2 * kw + core]
         for kh in range(3) for kw in range(3)], axis=0)
    y3 = jnp.dot(w3_ref[...], st3, preferred_element_type=jnp.float32)
    y3 = jnp.maximum(y3 + b3_ref[...], 0.0)
    mh = jnp.dot(w4_ref[...], y3.astype(_BF),
                 preferred_element_type=jnp.float32)
    o_ref[...] = _pad_lanes(mh.astype(_BF), o_ref.shape[-1])


def _adapt_body(x_ref, wa_ref, mhat_ref, sh_ref, sw_ref, b4_ref, o_ref,
                ext_ref, *, c, wp, h, w, core):
    h2 = h // 2
    acc = jnp.zeros((h, wp), jnp.float32)
    for t in range(9):
        tmp = jnp.dot(mhat_ref[t * h2:(t + 1) * h2, :], sw_ref[t],
                      preferred_element_type=jnp.float32)
        acc = acc + jnp.dot(sh_ref[t], tmp,
                            preferred_element_type=jnp.float32)
    mask2d = jax.nn.sigmoid(acc + b4_ref[...])
    mask_flat = jnp.concatenate([mask2d[r:r + 1, :] for r in range(h)],
                                axis=1)
    ext_ref[...] = jnp.zeros(ext_ref.shape, _BF)
    xb = x_ref[...].astype(_BF)
    for r in range(h):
        base = 2 * wp + r * wp + 1
        ext_ref[:, base:base + w] = xb[:, r * w:(r + 1) * w]
    st = _taps(lambda o, m: ext_ref[:, o:o + m], wp, core)
    adapted = jnp.dot(wa_ref[...], st, preferred_element_type=jnp.float32)
    am = adapted * mask_flat
    for r in range(h):
        o_ref[:, r * w:(r + 1) * w] = x_ref[:, r * w:(r + 1) * w] \
            + am[:, r * wp + 1:r * wp + 1 + w]


def kernel(m1_w, m1_b, bn1_gamma, bn1_beta, bn1_mean, bn1_var,
           m2_w, m2_b, bn2_gamma, bn2_beta, bn2_mean, bn2_var,
           m3_w, m3_b, bn3_gamma, bn3_beta, bn3_mean, bn3_var,
           m4_w, m4_b, bn4_gamma, bn4_beta, bn4_mean, bn4_var,
           r_w1, r_b1, r_w2, r_b2, weight_pool, x):
    x = x.astype(jnp.float32)
    n, c, h, w = x.shape
    h2, w2 = h // 2, w // 2
    wp = w + 2
    wph = wp // 2
    core = h * wp
    lh = h2 * wp
    core_pad = _rup(core, 128)
    le_pad = _rup((h + 4) * wp, 128)
    ls_pad = _rup((h2 + 4) * wp, 128)
    lsp_pad = _rup((h + 4) * wph + 1, 128)
    lmid_pad = _rup(lh, 128)

    w1s, b1 = _fold_conv_bn(m1_w, m1_b, bn1_gamma, bn1_beta, bn1_mean, bn1_var)
    w2s, b2 = _fold_conv_bn(m2_w, m2_b, bn2_gamma, bn2_beta, bn2_mean, bn2_var)
    w3s, b3 = _fold_conv_bn(m3_w, m3_b, bn3_gamma, bn3_beta, bn3_mean, bn3_var)
    s4 = bn4_gamma * jax.lax.rsqrt(bn4_var + _EPS)
    w4f = m4_w * s4[:, None, None, None]
    b4 = (m4_b - bn4_mean) * s4 + bn4_beta
    w4m = jnp.transpose(w4f[0], (1, 2, 0)).reshape(9, 16)
    w4p = jnp.zeros((16, 16), jnp.float32).at[:9].set(w4m).astype(_BF)

    s = jnp.array([[1.0 / 2.0, 1.0 / 1.5]], jnp.float32)
    hid = jax.nn.relu(s @ r_w1 + r_b1)
    rw = jax.nn.softmax(hid @ r_w2 + r_b2, axis=-1)[0]
    fused = jnp.einsum("e,eoikl->oikl", rw, weight_pool)
    was = jnp.transpose(fused, (0, 2, 3, 1)).reshape(c, 9 * c).astype(_BF)

    vme_np = np.zeros((h, wp), np.float32)
    vme_np[0:h:2, 1:2 * w2:2] = 1.0
    vme = jnp.asarray(vme_np.reshape(1, core))
    sh, sw = _mask_mats(h, w)

    cparams = pltpu.CompilerParams(dimension_semantics=("parallel",),
                                   vmem_limit_bytes=64 * 1024 * 1024)

    mh = pl.pallas_call(
        functools.partial(_branch_body, c=c, wp=wp, h=h, w=w, core=core),
        out_shape=jax.ShapeDtypeStruct((n, 16, core), _BF),
        grid=(n,),
        in_specs=[
            pl.BlockSpec((None, c, h * w), lambda i: (i, 0, 0)),
            pl.BlockSpec((16, 9 * c), lambda i: (0, 0)),
            pl.BlockSpec((16, 1), lambda i: (0, 0)),
            pl.BlockSpec((16, 144), lambda i: (0, 0)),
            pl.BlockSpec((16, 1), lambda i: (0, 0)),
            pl.BlockSpec((16, 144), lambda i: (0, 0)),
            pl.BlockSpec((16, 1), lambda i: (0, 0)),
            pl.BlockSpec((16, 16), lambda i: (0, 0)),
            pl.BlockSpec((1, core), lambda i: (0, 0)),
        ],
        out_specs=pl.BlockSpec((None, 16, core), lambda i: (i, 0, 0)),
        scratch_shapes=[pltpu.VMEM((c, _rup(core + 4 * wp, 128)), _BF)],
        compiler_params=cparams,
    )(x.reshape(n, c, h * w), w1s.astype(_BF), b1.reshape(16, 1),
      w2s.astype(_BF), b2.reshape(16, 1), w3s.astype(_BF),
      b3.reshape(16, 1), w4p, vme)

    mhat = (mh[:, :9, :].reshape(n, 9, h, wp)[:, :, ::2, 1::2][..., :w2]
            .reshape(n, 9 * h2, w2).astype(jnp.float32))

    outflat = pl.pallas_call(
        functools.partial(_adapt_body, c=c, wp=wp, h=h, w=w, core=core),
        out_shape=jax.ShapeDtypeStruct((n, c, h * w), jnp.float32),
        grid=(n,),
        in_specs=[
            pl.BlockSpec((None, c, h * w), lambda i: (i, 0, 0)),
            pl.BlockSpec((c, 9 * c), lambda i: (0, 0)),
            pl.BlockSpec((None, 9 * h2, w2), lambda i: (i, 0, 0)),
            pl.BlockSpec((9, h, h2), lambda i: (0, 0, 0)),
            pl.BlockSpec((9, w2, wp), lambda i: (0, 0, 0)),
            pl.BlockSpec((1, 1), lambda i: (0, 0)),
        ],
        out_specs=pl.BlockSpec((None, c, h * w), lambda i: (i, 0, 0)),
        scratch_shapes=[pltpu.VMEM((c, _rup(core + 4 * wp, 128)), _BF)],
        compiler_params=cparams,
    )(x.reshape(n, c, h * w), was, mhat, sh, sw, b4.reshape(1, 1))

    return outflat.reshape(n, c, h, w)
